# Initial kernel scaffold; baseline (speedup 1.0000x reference)
import functools

import jax
import jax.numpy as jnp
from jax import lax
from jax.experimental import pallas as pl
from jax.experimental.pallas import tpu as pltpu
from jax.experimental.pallas import tpu_sc as plsc

N = 10000
NPAD = 10240
E = 320000
D_IN = 128
D_HID = 64
D_OUT = 32

NC = 2
NS = 16
NW = NC * NS
CK = 128
NCH = 80
E_PER_TILE = NCH * CK
EPAD = NW * E_PER_TILE
ROWS_PER_TILE = NPAD // NS

_MESH = plsc.VectorSubcoreMesh(core_axis_name="c", subcore_axis_name="s",
                               num_cores=NC, num_subcores=NS)


def _zero_vmem_1d(ref, n):
    def body(i, _):
        ref[pl.ds(i * 16, 16)] = jnp.zeros((16,), jnp.float32)
        return 0
    lax.fori_loop(0, n // 16, body, 0)


def _zero_vmem_2d(ref, rows, cols):
    per_row = cols // 16
    def body(i, _):
        r = i // per_row
        k = (i % per_row) * 16
        ref[r, pl.ds(k, 16)] = jnp.zeros((16,), jnp.float32)
        return 0
    lax.fori_loop(0, rows * per_row, body, 0)


@functools.partial(
    pl.kernel,
    out_type=jax.ShapeDtypeStruct((NC, NPAD), jnp.float32),
    mesh=_MESH,
    scratch_types=[
        pltpu.VMEM((NCH, CK), jnp.int32),
        pltpu.VMEM((CK,), jnp.float32),
        pltpu.VMEM((ROWS_PER_TILE,), jnp.float32),
        pltpu.VMEM_SHARED((NPAD,), jnp.float32),
    ],
)
def _deg(dst_hbm, out_hbm, didx, ones_v, zbuf, acc):
    c = lax.axis_index("c")
    s = lax.axis_index("s")
    wid = c * NS + s

    pltpu.sync_copy(dst_hbm.at[wid], didx)

    _zero_vmem_1d(zbuf, ROWS_PER_TILE)
    def fill_ones(i, _):
        ones_v[pl.ds(i * 16, 16)] = jnp.ones((16,), jnp.float32)
        return 0
    lax.fori_loop(0, CK // 16, fill_ones, 0)

    pltpu.sync_copy(zbuf, acc.at[pl.ds(s * ROWS_PER_TILE, ROWS_PER_TILE)])
    plsc.subcore_barrier()

    def body(j, _):
        pltpu.sync_copy(ones_v, acc.at[didx.at[j]], add=True)
        return 0
    lax.fori_loop(0, NCH, body, 0)

    plsc.subcore_barrier()
    pltpu.sync_copy(acc.at[pl.ds(s * ROWS_PER_TILE, ROWS_PER_TILE)],
                    out_hbm.at[c, pl.ds(s * ROWS_PER_TILE, ROWS_PER_TILE)])


def _make_propagate(D):
    @functools.partial(
        pl.kernel,
        out_type=jax.ShapeDtypeStruct((NC, NPAD, D), jnp.float32),
        mesh=_MESH,
        scratch_types=[
            pltpu.VMEM((NCH, CK), jnp.int32),
            pltpu.VMEM((NCH, CK), jnp.int32),
            pltpu.VMEM((CK, D), jnp.float32),
            pltpu.VMEM((CK, D), jnp.float32),
            pltpu.VMEM_SHARED((NPAD, D), jnp.float32),
            pltpu.SemaphoreType.DMA,
            pltpu.SemaphoreType.DMA,
        ],
    )
    def prop(src_hbm, dst_hbm, h_hbm, out_hbm, sidx, didx, buf0, buf1, acc,
             gsem0, gsem1):
        c = lax.axis_index("c")
        s = lax.axis_index("s")
        wid = c * NS + s

        pltpu.sync_copy(src_hbm.at[wid], sidx)
        pltpu.sync_copy(dst_hbm.at[wid], didx)

        _zero_vmem_2d(buf0, CK, D)
        def zcopy(r, _):
            pltpu.sync_copy(buf0,
                            acc.at[pl.ds(s * ROWS_PER_TILE + r * CK, CK)])
            return 0
        lax.fori_loop(0, ROWS_PER_TILE // CK, zcopy, 0)
        plsc.subcore_barrier()

        pltpu.async_copy(h_hbm.at[sidx.at[0]], buf0, gsem0)

        def body(g, _):
            j0 = 2 * g
            pltpu.async_copy(h_hbm.at[sidx.at[j0 + 1]], buf1, gsem1)
            pltpu.make_async_copy(h_hbm.at[sidx.at[j0]], buf0, gsem0).wait()
            pltpu.sync_copy(buf0, acc.at[didx.at[j0]], add=True)

            @pl.when(g < NCH // 2 - 1)
            def _():
                pltpu.async_copy(h_hbm.at[sidx.at[j0 + 2]], buf0, gsem0)

            pltpu.make_async_copy(h_hbm.at[sidx.at[j0 + 1]], buf1,
                                  gsem1).wait()
            pltpu.sync_copy(buf1, acc.at[didx.at[j0 + 1]], add=True)
            return 0
        lax.fori_loop(0, NCH // 2, body, 0)

        plsc.subcore_barrier()
        pltpu.sync_copy(
            acc.at[pl.ds(s * ROWS_PER_TILE, ROWS_PER_TILE)],
            out_hbm.at[c, pl.ds(s * ROWS_PER_TILE, ROWS_PER_TILE), :])

    return prop


_prop_hid = _make_propagate(D_HID)
_prop_out = _make_propagate(D_OUT)


_R = 2048


def _k1_body(x_ref, w_ref, d0_ref, d1_ref, h_ref, dis_ref):
    deg = d0_ref[...] + d1_ref[...] + 1.0
    dis = lax.rsqrt(deg)
    h = jnp.dot(x_ref[...], w_ref[...], preferred_element_type=jnp.float32)
    h_ref[...] = h * dis
    dis_ref[...] = dis


_k1 = pl.pallas_call(
    _k1_body,
    grid=(NPAD // _R,),
    in_specs=[
        pl.BlockSpec((_R, D_IN), lambda i: (i, 0)),
        pl.BlockSpec((D_IN, D_HID), lambda i: (0, 0)),
        pl.BlockSpec((_R, 1), lambda i: (i, 0)),
        pl.BlockSpec((_R, 1), lambda i: (i, 0)),
    ],
    out_specs=[
        pl.BlockSpec((_R, D_HID), lambda i: (i, 0)),
        pl.BlockSpec((_R, 1), lambda i: (i, 0)),
    ],
    out_shape=[
        jax.ShapeDtypeStruct((NPAD, D_HID), jnp.float32),
        jax.ShapeDtypeStruct((NPAD, 1), jnp.float32),
    ],
)


def _k2_body(p0_ref, p1_ref, h_ref, dis_ref, b_ref, w_ref, o_ref):
    dis = dis_ref[...]
    z = jnp.maximum(
        (p0_ref[...] + p1_ref[...] + h_ref[...]) * dis + b_ref[...], 0.0)
    o_ref[...] = jnp.dot(z, w_ref[...],
                         preferred_element_type=jnp.float32) * dis


_k2 = pl.pallas_call(
    _k2_body,
    grid=(NPAD // _R,),
    in_specs=[
        pl.BlockSpec((_R, D_HID), lambda i: (i, 0)),
        pl.BlockSpec((_R, D_HID), lambda i: (i, 0)),
        pl.BlockSpec((_R, D_HID), lambda i: (i, 0)),
        pl.BlockSpec((_R, 1), lambda i: (i, 0)),
        pl.BlockSpec((1, D_HID), lambda i: (0, 0)),
        pl.BlockSpec((D_HID, D_OUT), lambda i: (0, 0)),
    ],
    out_specs=pl.BlockSpec((_R, D_OUT), lambda i: (i, 0)),
    out_shape=jax.ShapeDtypeStruct((NPAD, D_OUT), jnp.float32),
)


def _k3_body(p0_ref, p1_ref, h_ref, dis_ref, b_ref, o_ref):
    z = (p0_ref[...] + p1_ref[...] + h_ref[...]) * dis_ref[...] + b_ref[...]
    o_ref[...] = jnp.maximum(z, 0.0)


_k3 = pl.pallas_call(
    _k3_body,
    grid=(NPAD // _R,),
    in_specs=[
        pl.BlockSpec((_R, D_OUT), lambda i: (i, 0)),
        pl.BlockSpec((_R, D_OUT), lambda i: (i, 0)),
        pl.BlockSpec((_R, D_OUT), lambda i: (i, 0)),
        pl.BlockSpec((_R, 1), lambda i: (i, 0)),
        pl.BlockSpec((1, D_OUT), lambda i: (0, 0)),
    ],
    out_specs=pl.BlockSpec((_R, D_OUT), lambda i: (i, 0)),
    out_shape=jax.ShapeDtypeStruct((NPAD, D_OUT), jnp.float32),
)


def kernel(x, edge_index, W1, b1, W2, b2):
    src = edge_index[0].astype(jnp.int32)
    dst = edge_index[1].astype(jnp.int32)
    pad_idx = N + (jnp.arange(EPAD - E, dtype=jnp.int32) % (NPAD - N))
    srcp = jnp.concatenate([src, pad_idx]).reshape(NW, NCH, CK)
    dstp = jnp.concatenate([dst, pad_idx]).reshape(NW, NCH, CK)
    xp = jnp.pad(x, ((0, NPAD - N), (0, 0)))

    degp = _deg(dstp)
    d0 = degp[0].reshape(NPAD, 1)
    d1 = degp[1].reshape(NPAD, 1)

    h1, dis = _k1(xp, W1, d0, d1)
    s1 = _prop_hid(srcp, dstp, h1)
    h2 = _k2(s1[0], s1[1], h1, dis, b1.reshape(1, D_HID), W2)
    s2 = _prop_out(srcp, dstp, h2)
    z2 = _k3(s2[0], s2[1], h2, dis, b2.reshape(1, D_OUT))
    return z2[:N]


# speedup vs baseline: 39.5806x; 39.5806x over previous
import functools

import jax
import jax.numpy as jnp
from jax import lax
from jax.experimental import pallas as pl
from jax.experimental.pallas import tpu as pltpu
from jax.experimental.pallas import tpu_sc as plsc

N = 10000
NPAD = 10240
E = 320000
D_IN = 128
D_HID = 64
D_OUT = 32

NC = 2
NS = 16
NW = NC * NS
CK = 128
NCH = 80
E_PER_TILE = NCH * CK
EPAD = NW * E_PER_TILE
ROWS_PER_TILE = NPAD // NS

_MESH = plsc.VectorSubcoreMesh(core_axis_name="c", subcore_axis_name="s",
                               num_cores=NC, num_subcores=NS)


def _zero_vmem_1d(ref, n):
    def body(i, _):
        ref[pl.ds(i * 16, 16)] = jnp.zeros((16,), jnp.float32)
        return 0
    lax.fori_loop(0, n // 16, body, 0)


def _zero_vmem_2d(ref, rows, cols):
    per_row = cols // 16
    def body(i, _):
        r = i // per_row
        k = (i % per_row) * 16
        ref[r, pl.ds(k, 16)] = jnp.zeros((16,), jnp.float32)
        return 0
    lax.fori_loop(0, rows * per_row, body, 0)


@functools.partial(
    pl.kernel,
    out_type=jax.ShapeDtypeStruct((NC, NPAD), jnp.float32),
    mesh=_MESH,
    scratch_types=[
        pltpu.VMEM((NCH, CK), jnp.int32),
        pltpu.VMEM((CK,), jnp.float32),
        pltpu.VMEM((ROWS_PER_TILE,), jnp.float32),
        pltpu.VMEM_SHARED((NPAD,), jnp.float32),
    ],
)
def _deg(dst_hbm, out_hbm, didx, ones_v, zbuf, acc):
    c = lax.axis_index("c")
    s = lax.axis_index("s")
    wid = c * NS + s

    pltpu.sync_copy(dst_hbm.at[wid], didx)

    _zero_vmem_1d(zbuf, ROWS_PER_TILE)
    def fill_ones(i, _):
        ones_v[pl.ds(i * 16, 16)] = jnp.ones((16,), jnp.float32)
        return 0
    lax.fori_loop(0, CK // 16, fill_ones, 0)

    pltpu.sync_copy(zbuf, acc.at[pl.ds(s * ROWS_PER_TILE, ROWS_PER_TILE)])
    plsc.subcore_barrier()

    def body(j, _):
        pltpu.sync_copy(ones_v, acc.at[didx.at[j]], add=True)
        return 0
    lax.fori_loop(0, NCH, body, 0)

    plsc.subcore_barrier()
    pltpu.sync_copy(acc.at[pl.ds(s * ROWS_PER_TILE, ROWS_PER_TILE)],
                    out_hbm.at[c, pl.ds(s * ROWS_PER_TILE, ROWS_PER_TILE)])


def _make_propagate(D):
    @functools.partial(
        pl.kernel,
        out_type=jax.ShapeDtypeStruct((NC, NPAD, D), jnp.float32),
        mesh=_MESH,
        scratch_types=[
            pltpu.VMEM((NCH, CK), jnp.int32),
            pltpu.VMEM((NCH, CK), jnp.int32),
            pltpu.VMEM((CK, D), jnp.float32),
            pltpu.VMEM((CK, D), jnp.float32),
            pltpu.VMEM_SHARED((NPAD, D), jnp.float32),
            pltpu.SemaphoreType.DMA,
            pltpu.SemaphoreType.DMA,
        ],
        compiler_params=pltpu.CompilerParams(use_tc_tiling_on_sc=False),
    )
    def prop(src_hbm, dst_hbm, h_hbm, out_hbm, sidx, didx, buf0, buf1, acc,
             gsem0, gsem1):
        c = lax.axis_index("c")
        s = lax.axis_index("s")
        wid = c * NS + s

        pltpu.sync_copy(src_hbm.at[wid], sidx)
        pltpu.sync_copy(dst_hbm.at[wid], didx)

        _zero_vmem_2d(buf0, CK, D)
        def zcopy(r, _):
            pltpu.sync_copy(buf0,
                            acc.at[pl.ds(s * ROWS_PER_TILE + r * CK, CK)])
            return 0
        lax.fori_loop(0, ROWS_PER_TILE // CK, zcopy, 0)
        plsc.subcore_barrier()

        pltpu.async_copy(h_hbm.at[sidx.at[0]], buf0, gsem0)

        def body(g, _):
            j0 = 2 * g
            pltpu.async_copy(h_hbm.at[sidx.at[j0 + 1]], buf1, gsem1)
            pltpu.make_async_copy(h_hbm.at[sidx.at[j0]], buf0, gsem0).wait()
            pltpu.sync_copy(buf0, acc.at[didx.at[j0]], add=True)

            @pl.when(g < NCH // 2 - 1)
            def _():
                pltpu.async_copy(h_hbm.at[sidx.at[j0 + 2]], buf0, gsem0)

            pltpu.make_async_copy(h_hbm.at[sidx.at[j0 + 1]], buf1,
                                  gsem1).wait()
            pltpu.sync_copy(buf1, acc.at[didx.at[j0 + 1]], add=True)
            return 0
        lax.fori_loop(0, NCH // 2, body, 0)

        plsc.subcore_barrier()
        pltpu.sync_copy(
            acc.at[pl.ds(s * ROWS_PER_TILE, ROWS_PER_TILE)],
            out_hbm.at[c, pl.ds(s * ROWS_PER_TILE, ROWS_PER_TILE), :])

    return prop


_prop_hid = _make_propagate(D_HID)
_prop_out = _make_propagate(D_OUT)


_R = 2048


def _k1_body(x_ref, w_ref, d0_ref, d1_ref, h_ref, dis_ref):
    deg = d0_ref[...] + d1_ref[...] + 1.0
    dis = lax.rsqrt(deg)
    h = jnp.dot(x_ref[...], w_ref[...], preferred_element_type=jnp.float32)
    h_ref[...] = h * dis
    dis_ref[...] = dis


_k1 = pl.pallas_call(
    _k1_body,
    grid=(NPAD // _R,),
    in_specs=[
        pl.BlockSpec((_R, D_IN), lambda i: (i, 0)),
        pl.BlockSpec((D_IN, D_HID), lambda i: (0, 0)),
        pl.BlockSpec((_R, 1), lambda i: (i, 0)),
        pl.BlockSpec((_R, 1), lambda i: (i, 0)),
    ],
    out_specs=[
        pl.BlockSpec((_R, D_HID), lambda i: (i, 0)),
        pl.BlockSpec((_R, 1), lambda i: (i, 0)),
    ],
    out_shape=[
        jax.ShapeDtypeStruct((NPAD, D_HID), jnp.float32),
        jax.ShapeDtypeStruct((NPAD, 1), jnp.float32),
    ],
)


def _k2_body(p0_ref, p1_ref, h_ref, dis_ref, b_ref, w_ref, o_ref):
    dis = dis_ref[...]
    z = jnp.maximum(
        (p0_ref[...] + p1_ref[...] + h_ref[...]) * dis + b_ref[...], 0.0)
    o_ref[...] = jnp.dot(z, w_ref[...],
                         preferred_element_type=jnp.float32) * dis


_k2 = pl.pallas_call(
    _k2_body,
    grid=(NPAD // _R,),
    in_specs=[
        pl.BlockSpec((_R, D_HID), lambda i: (i, 0)),
        pl.BlockSpec((_R, D_HID), lambda i: (i, 0)),
        pl.BlockSpec((_R, D_HID), lambda i: (i, 0)),
        pl.BlockSpec((_R, 1), lambda i: (i, 0)),
        pl.BlockSpec((1, D_HID), lambda i: (0, 0)),
        pl.BlockSpec((D_HID, D_OUT), lambda i: (0, 0)),
    ],
    out_specs=pl.BlockSpec((_R, D_OUT), lambda i: (i, 0)),
    out_shape=jax.ShapeDtypeStruct((NPAD, D_OUT), jnp.float32),
)


def _k3_body(p0_ref, p1_ref, h_ref, dis_ref, b_ref, o_ref):
    z = (p0_ref[...] + p1_ref[...] + h_ref[...]) * dis_ref[...] + b_ref[...]
    o_ref[...] = jnp.maximum(z, 0.0)


_k3 = pl.pallas_call(
    _k3_body,
    grid=(NPAD // _R,),
    in_specs=[
        pl.BlockSpec((_R, D_OUT), lambda i: (i, 0)),
        pl.BlockSpec((_R, D_OUT), lambda i: (i, 0)),
        pl.BlockSpec((_R, D_OUT), lambda i: (i, 0)),
        pl.BlockSpec((_R, 1), lambda i: (i, 0)),
        pl.BlockSpec((1, D_OUT), lambda i: (0, 0)),
    ],
    out_specs=pl.BlockSpec((_R, D_OUT), lambda i: (i, 0)),
    out_shape=jax.ShapeDtypeStruct((NPAD, D_OUT), jnp.float32),
)


def kernel(x, edge_index, W1, b1, W2, b2):
    src = edge_index[0].astype(jnp.int32)
    dst = edge_index[1].astype(jnp.int32)
    pad_idx = N + (jnp.arange(EPAD - E, dtype=jnp.int32) % (NPAD - N))
    srcp = jnp.concatenate([src, pad_idx]).reshape(NW, NCH, CK)
    dstp = jnp.concatenate([dst, pad_idx]).reshape(NW, NCH, CK)
    xp = jnp.pad(x, ((0, NPAD - N), (0, 0)))

    degp = _deg(dstp)
    d0 = degp[0].reshape(NPAD, 1)
    d1 = degp[1].reshape(NPAD, 1)

    h1, dis = _k1(xp, W1, d0, d1)
    s1 = _prop_hid(srcp, dstp, h1)
    h2 = _k2(s1[0], s1[1], h1, dis, b1.reshape(1, D_HID), W2)
    s2 = _prop_out(srcp, dstp, h2)
    z2 = _k3(s2[0], s2[1], h2, dis, b2.reshape(1, D_OUT))
    return z2[:N]
